# baseline (speedup 1.0000x reference)
import functools

import jax
import jax.numpy as jnp
import numpy as np
from jax.experimental import pallas as pl
from jax.experimental.pallas import tpu as pltpu

_F = 128
_STRIDE = 16
_BH = 32
_NBLK = _F // _BH
_M = _BH * _F


def _anchor_wh():
    size = 16 * 16
    ratios = np.array([0.5, 1.0, 2.0])
    scales = np.array([8, 16, 32])
    ws = np.round(np.sqrt(size / ratios))
    hs = np.round(ws * ratios)
    aw = np.empty(9)
    ah = np.empty(9)
    k = 0
    for w, h in zip(ws, hs):
        for s in scales:
            aw[k] = w * s
            ah[k] = h * s
            k += 1
    return aw, ah


_AW, _AH = _anchor_wh()

_LANE_MUL = np.concatenate([_AW, _AH, np.ones(9), np.ones(9), np.ones(9)])
_LANE_ADD = np.concatenate([np.zeros(9), np.zeros(9), _AW, _AH, np.zeros(9)])


def _rpn_kernel(xa_ref, xb_ref, w9_ref, bb_ref, hw_ref, hb_ref,
                lm_ref, la_ref, out_ref, y_acc):
    i = pl.program_id(0)

    first = True
    for dy in range(3):
        for dx in range(3):
            wk = w9_ref[dy * 3 + dx]
            rows = _BH - dy
            lhs = xa_ref[dy:_BH, dx:dx + _F, :].reshape(rows * _F, 1024)
            contrib = jnp.dot(lhs, wk, preferred_element_type=jnp.float32)
            if first:
                y_acc[0:rows * _F, :] = contrib
                first = False
            else:
                y_acc[0:rows * _F, :] += contrib
            if dy > 0:
                lhs2 = xb_ref[0:dy, dx:dx + _F, :].reshape(dy * _F, 1024)
                y_acc[rows * _F:_M, :] += jnp.dot(
                    lhs2, wk, preferred_element_type=jnp.float32)

    y = jnp.maximum(y_acc[...] + bb_ref[...], 0.0).astype(jnp.bfloat16)
    hd = jnp.dot(y, hw_ref[...], preferred_element_type=jnp.float32)
    hd = hd + hb_ref[...]

    mi = jax.lax.broadcasted_iota(jnp.int32, (_M, 1), 0)
    ax = (i * (_BH * _STRIDE) + (mi // _F) * _STRIDE).astype(jnp.float32)
    ay = ((mi % _F) * _STRIDE).astype(jnp.float32)
    ci = jax.lax.broadcasted_iota(jnp.int32, (_M, 45), 1)

    t = jnp.where((ci >= 18) & (ci < 36), jnp.exp(hd), hd)
    t = jnp.where(ci >= 36, jax.nn.sigmoid(hd), t)
    pos = jnp.where(ci < 9, ax, jnp.where(ci < 18, ay, 0.0))
    out_ref[...] = lm_ref[...] * t + la_ref[...] + pos


@functools.partial(jax.jit, static_argnames=())
def kernel(x, base_w, base_b, cls_w, cls_b, reg_w, reg_b):
    F = _F
    xt = jnp.transpose(x[0], (1, 2, 0))
    xp = jnp.pad(xt, ((1, _BH - 1), (1, 1), (0, 0)))
    xp = xp.astype(jnp.float8_e4m3fn)

    w9 = jnp.transpose(base_w, (2, 3, 1, 0)).reshape(9, 1024, 512)
    w9 = (w9 * 64.0).astype(jnp.float8_e4m3fn)
    bb = (base_b * 64.0).reshape(1, 512).astype(jnp.float32)

    reg_ws = reg_w[:, :, 0, 0]
    cls_ws = cls_w[:, :, 0, 0]
    reg_part = reg_ws.reshape(9, 4, 512).transpose(1, 0, 2).reshape(36, 512)
    odd = np.arange(9) * 2 + 1
    partner = (odd + 9) % 18
    score_part = cls_ws[odd] - cls_ws[partner]
    hw = jnp.concatenate([reg_part, score_part], axis=0).T
    hw = (hw / 64.0).astype(jnp.bfloat16)

    reg_bias = reg_b.reshape(9, 4).transpose(1, 0).reshape(36)
    score_bias = cls_b[odd] - cls_b[partner]
    hb = jnp.concatenate([reg_bias, score_bias]).reshape(1, 45)
    hb = hb.astype(jnp.float32)

    lm = jnp.asarray(_LANE_MUL, dtype=jnp.float32).reshape(1, 45)
    la = jnp.asarray(_LANE_ADD, dtype=jnp.float32).reshape(1, 45)

    out = pl.pallas_call(
        _rpn_kernel,
        out_shape=jax.ShapeDtypeStruct((F * F, 45), jnp.float32),
        grid=(_NBLK,),
        in_specs=[
            pl.BlockSpec((_BH, F + 2, 1024), lambda i: (i, 0, 0)),
            pl.BlockSpec((_BH, F + 2, 1024), lambda i: (i + 1, 0, 0)),
            pl.BlockSpec((9, 1024, 512), lambda i: (0, 0, 0)),
            pl.BlockSpec((1, 512), lambda i: (0, 0)),
            pl.BlockSpec((512, 45), lambda i: (0, 0)),
            pl.BlockSpec((1, 45), lambda i: (0, 0)),
            pl.BlockSpec((1, 45), lambda i: (0, 0)),
            pl.BlockSpec((1, 45), lambda i: (0, 0)),
        ],
        out_specs=pl.BlockSpec((_M, 45), lambda i: (i, 0)),
        scratch_shapes=[pltpu.VMEM((_M, 512), jnp.float32)],
        compiler_params=pltpu.CompilerParams(
            dimension_semantics=("parallel",),
            vmem_limit_bytes=52 * 1024 * 1024,
        ),
        name="rpn_fused_fp8",
    )(xp, xp, w9, bb, hw, hb, lm, la)

    return out.reshape(F * F, 5, 9).transpose(0, 2, 1).reshape(-1, 5)


# speedup vs baseline: 1.3859x; 1.3859x over previous
import functools

import jax
import jax.numpy as jnp
import numpy as np
from jax.experimental import pallas as pl
from jax.experimental.pallas import tpu as pltpu

_F = 128
_STRIDE = 16
_BH = 8
_NBLK = _F // _BH
_M = _BH * _F


def _anchor_wh():
    size = 16 * 16
    ratios = np.array([0.5, 1.0, 2.0])
    scales = np.array([8, 16, 32])
    ws = np.round(np.sqrt(size / ratios))
    hs = np.round(ws * ratios)
    aw = np.empty(9)
    ah = np.empty(9)
    k = 0
    for w, h in zip(ws, hs):
        for s in scales:
            aw[k] = w * s
            ah[k] = h * s
            k += 1
    return aw, ah


_AW, _AH = _anchor_wh()

_LANE_MUL = np.concatenate([_AW, _AH, np.ones(9), np.ones(9), np.ones(9)])
_LANE_ADD = np.concatenate([np.zeros(9), np.zeros(9), _AW, _AH, np.zeros(9)])


def _rpn_kernel(xa_ref, xb_ref, w9_ref, bb_ref, hw_ref, hb_ref,
                lm_ref, la_ref, out_ref, y_acc):
    i = pl.program_id(0)

    first = True
    for dy in range(3):
        for dx in range(3):
            wk = w9_ref[dy * 3 + dx]
            rows = _BH - dy
            lhs = xa_ref[dy:_BH, dx:dx + _F, :].reshape(rows * _F, 1024)
            contrib = jnp.dot(lhs, wk, preferred_element_type=jnp.float32)
            if first:
                y_acc[0:rows * _F, :] = contrib
                first = False
            else:
                y_acc[0:rows * _F, :] += contrib
            if dy > 0:
                lhs2 = xb_ref[0:dy, dx:dx + _F, :].reshape(dy * _F, 1024)
                y_acc[rows * _F:_M, :] += jnp.dot(
                    lhs2, wk, preferred_element_type=jnp.float32)

    y = jnp.maximum(y_acc[...] + bb_ref[...], 0.0).astype(jnp.bfloat16)
    hd = jnp.dot(y, hw_ref[...], preferred_element_type=jnp.float32)
    hd = hd + hb_ref[...]

    mi = jax.lax.broadcasted_iota(jnp.int32, (_M, 1), 0)
    ax = (i * (_BH * _STRIDE) + (mi // _F) * _STRIDE).astype(jnp.float32)
    ay = ((mi % _F) * _STRIDE).astype(jnp.float32)
    ci = jax.lax.broadcasted_iota(jnp.int32, (_M, 45), 1)

    t = jnp.where((ci >= 18) & (ci < 36), jnp.exp(hd), hd)
    t = jnp.where(ci >= 36, jax.nn.sigmoid(hd), t)
    pos = jnp.where(ci < 9, ax, jnp.where(ci < 18, ay, 0.0))
    out_ref[...] = lm_ref[...] * t + la_ref[...] + pos


@functools.partial(jax.jit, static_argnames=())
def kernel(x, base_w, base_b, cls_w, cls_b, reg_w, reg_b):
    F = _F
    xt = jnp.transpose(x[0], (1, 2, 0))
    xp = jnp.pad(xt, ((1, _BH - 1), (1, 1), (0, 0)))
    xp = xp.astype(jnp.float8_e4m3fn)

    w9 = jnp.transpose(base_w, (2, 3, 1, 0)).reshape(9, 1024, 512)
    w9 = (w9 * 64.0).astype(jnp.float8_e4m3fn)
    bb = (base_b * 64.0).reshape(1, 512).astype(jnp.float32)

    reg_ws = reg_w[:, :, 0, 0]
    cls_ws = cls_w[:, :, 0, 0]
    reg_part = reg_ws.reshape(9, 4, 512).transpose(1, 0, 2).reshape(36, 512)
    odd = np.arange(9) * 2 + 1
    partner = (odd + 9) % 18
    score_part = cls_ws[odd] - cls_ws[partner]
    hw = jnp.concatenate([reg_part, score_part], axis=0).T
    hw = (hw / 64.0).astype(jnp.bfloat16)

    reg_bias = reg_b.reshape(9, 4).transpose(1, 0).reshape(36)
    score_bias = cls_b[odd] - cls_b[partner]
    hb = jnp.concatenate([reg_bias, score_bias]).reshape(1, 45)
    hb = hb.astype(jnp.float32)

    lm = jnp.asarray(_LANE_MUL, dtype=jnp.float32).reshape(1, 45)
    la = jnp.asarray(_LANE_ADD, dtype=jnp.float32).reshape(1, 45)

    out = pl.pallas_call(
        _rpn_kernel,
        out_shape=jax.ShapeDtypeStruct((F * F, 45), jnp.float32),
        grid=(_NBLK,),
        in_specs=[
            pl.BlockSpec((_BH, F + 2, 1024), lambda i: (i, 0, 0)),
            pl.BlockSpec((_BH, F + 2, 1024), lambda i: (i + 1, 0, 0)),
            pl.BlockSpec((9, 1024, 512), lambda i: (0, 0, 0)),
            pl.BlockSpec((1, 512), lambda i: (0, 0)),
            pl.BlockSpec((512, 45), lambda i: (0, 0)),
            pl.BlockSpec((1, 45), lambda i: (0, 0)),
            pl.BlockSpec((1, 45), lambda i: (0, 0)),
            pl.BlockSpec((1, 45), lambda i: (0, 0)),
        ],
        out_specs=pl.BlockSpec((_M, 45), lambda i: (i, 0)),
        scratch_shapes=[pltpu.VMEM((_M, 512), jnp.float32)],
        compiler_params=pltpu.CompilerParams(
            dimension_semantics=("parallel",),
            vmem_limit_bytes=52 * 1024 * 1024,
        ),
        name="rpn_fused_fp8",
    )(xp, xp, w9, bb, hw, hb, lm, la)

    return out.reshape(F * F, 5, 9).transpose(0, 2, 1).reshape(-1, 5)


# speedup vs baseline: 1.3991x; 1.0095x over previous
import functools

import jax
import jax.numpy as jnp
import numpy as np
from jax.experimental import pallas as pl
from jax.experimental.pallas import tpu as pltpu

_F = 128
_STRIDE = 16
_BH = 4
_NBLK = _F // _BH
_M = _BH * _F


def _anchor_wh():
    size = 16 * 16
    ratios = np.array([0.5, 1.0, 2.0])
    scales = np.array([8, 16, 32])
    ws = np.round(np.sqrt(size / ratios))
    hs = np.round(ws * ratios)
    aw = np.empty(9)
    ah = np.empty(9)
    k = 0
    for w, h in zip(ws, hs):
        for s in scales:
            aw[k] = w * s
            ah[k] = h * s
            k += 1
    return aw, ah


_AW, _AH = _anchor_wh()

_LANE_MUL = np.concatenate([_AW, _AH, np.ones(9), np.ones(9), np.ones(9)])
_LANE_ADD = np.concatenate([np.zeros(9), np.zeros(9), _AW, _AH, np.zeros(9)])


def _rpn_kernel(xa_ref, xb_ref, w9_ref, bb_ref, hw_ref, hb_ref,
                lm_ref, la_ref, out_ref, y_acc):
    i = pl.program_id(0)

    first = True
    for dy in range(3):
        for dx in range(3):
            wk = w9_ref[dy * 3 + dx]
            rows = _BH - dy
            lhs = xa_ref[dy:_BH, dx:dx + _F, :].reshape(rows * _F, 1024)
            contrib = jnp.dot(lhs, wk, preferred_element_type=jnp.float32)
            if first:
                y_acc[0:rows * _F, :] = contrib
                first = False
            else:
                y_acc[0:rows * _F, :] += contrib
            if dy > 0:
                lhs2 = xb_ref[0:dy, dx:dx + _F, :].reshape(dy * _F, 1024)
                y_acc[rows * _F:_M, :] += jnp.dot(
                    lhs2, wk, preferred_element_type=jnp.float32)

    y = jnp.maximum(y_acc[...] + bb_ref[...], 0.0).astype(jnp.bfloat16)
    hd = jnp.dot(y, hw_ref[...], preferred_element_type=jnp.float32)
    hd = hd + hb_ref[...]

    mi = jax.lax.broadcasted_iota(jnp.int32, (_M, 1), 0)
    ax = (i * (_BH * _STRIDE) + (mi // _F) * _STRIDE).astype(jnp.float32)
    ay = ((mi % _F) * _STRIDE).astype(jnp.float32)
    ci = jax.lax.broadcasted_iota(jnp.int32, (_M, 45), 1)

    t = jnp.where((ci >= 18) & (ci < 36), jnp.exp(hd), hd)
    t = jnp.where(ci >= 36, jax.nn.sigmoid(hd), t)
    pos = jnp.where(ci < 9, ax, jnp.where(ci < 18, ay, 0.0))
    out_ref[...] = lm_ref[...] * t + la_ref[...] + pos


@functools.partial(jax.jit, static_argnames=())
def kernel(x, base_w, base_b, cls_w, cls_b, reg_w, reg_b):
    F = _F
    xt = jnp.transpose(x[0], (1, 2, 0))
    xp = jnp.pad(xt, ((1, _BH - 1), (1, 1), (0, 0)))
    xp = xp.astype(jnp.float8_e4m3fn)

    w9 = jnp.transpose(base_w, (2, 3, 1, 0)).reshape(9, 1024, 512)
    w9 = (w9 * 64.0).astype(jnp.float8_e4m3fn)
    bb = (base_b * 64.0).reshape(1, 512).astype(jnp.float32)

    reg_ws = reg_w[:, :, 0, 0]
    cls_ws = cls_w[:, :, 0, 0]
    reg_part = reg_ws.reshape(9, 4, 512).transpose(1, 0, 2).reshape(36, 512)
    odd = np.arange(9) * 2 + 1
    partner = (odd + 9) % 18
    score_part = cls_ws[odd] - cls_ws[partner]
    hw = jnp.concatenate([reg_part, score_part], axis=0).T
    hw = (hw / 64.0).astype(jnp.bfloat16)

    reg_bias = reg_b.reshape(9, 4).transpose(1, 0).reshape(36)
    score_bias = cls_b[odd] - cls_b[partner]
    hb = jnp.concatenate([reg_bias, score_bias]).reshape(1, 45)
    hb = hb.astype(jnp.float32)

    lm = jnp.asarray(_LANE_MUL, dtype=jnp.float32).reshape(1, 45)
    la = jnp.asarray(_LANE_ADD, dtype=jnp.float32).reshape(1, 45)

    out = pl.pallas_call(
        _rpn_kernel,
        out_shape=jax.ShapeDtypeStruct((F * F, 45), jnp.float32),
        grid=(_NBLK,),
        in_specs=[
            pl.BlockSpec((_BH, F + 2, 1024), lambda i: (i, 0, 0)),
            pl.BlockSpec((_BH, F + 2, 1024), lambda i: (i + 1, 0, 0)),
            pl.BlockSpec((9, 1024, 512), lambda i: (0, 0, 0)),
            pl.BlockSpec((1, 512), lambda i: (0, 0)),
            pl.BlockSpec((512, 45), lambda i: (0, 0)),
            pl.BlockSpec((1, 45), lambda i: (0, 0)),
            pl.BlockSpec((1, 45), lambda i: (0, 0)),
            pl.BlockSpec((1, 45), lambda i: (0, 0)),
        ],
        out_specs=pl.BlockSpec((_M, 45), lambda i: (i, 0)),
        scratch_shapes=[pltpu.VMEM((_M, 512), jnp.float32)],
        compiler_params=pltpu.CompilerParams(
            dimension_semantics=("parallel",),
            vmem_limit_bytes=52 * 1024 * 1024,
        ),
        name="rpn_fused_fp8",
    )(xp, xp, w9, bb, hw, hb, lm, la)

    return out.reshape(F * F, 5, 9).transpose(0, 2, 1).reshape(-1, 5)
